# baseline (speedup 1.0000x reference)
import functools
import math

import jax
import jax.numpy as jnp
import numpy as np
from jax import lax
from jax.experimental import pallas as pl
from jax.experimental.pallas import tpu as pltpu
from jax.experimental.pallas import tpu_sc as plsc

_NUM_WORKERS = 32
_LANES = 16
_BL = 128
_NBUF = 2


def _positional_encoding(maxlen, dim):
    pos = jnp.arange(maxlen, dtype=jnp.float32)
    i = np.arange(dim)
    terms = jnp.asarray(1.0 / (10000.0 ** (2.0 * (i // 2) / float(dim))),
                        dtype=jnp.float32)
    pe_val = pos[:, None] * terms[None, :]
    pe = jnp.zeros((maxlen, dim), dtype=jnp.float32)
    pe = pe.at[:, 0::2].set(jnp.sin(pe_val[:, 0::2]))
    pe = pe.at[:, 1::2].set(jnp.cos(pe_val[:, 0::2]))
    return pe


@functools.partial(jax.jit, static_argnames=("b", "l"))
def _run(idx, W, pe, b, l):
    n_slabs = idx.shape[1]
    d = W.shape[1]
    dgs = d // 8
    scale = 1.0 / math.sqrt(float(b))
    segs = d // _LANES
    n_bg = b // _BL
    mesh = plsc.VectorSubcoreMesh(core_axis_name="c", subcore_axis_name="s")

    @functools.partial(
        pl.kernel,
        mesh=mesh,
        out_type=jax.ShapeDtypeStruct((l, dgs, n_bg, 8, _BL), jnp.float32),
        scratch_types=[
            pltpu.VMEM((n_slabs, _BL), jnp.int32),
            pltpu.VMEM((l, d), jnp.float32),
            pltpu.VMEM((_BL, d), jnp.float32),
            pltpu.VMEM((_BL, d), jnp.float32),
            pltpu.VMEM((dgs, 8, _BL), jnp.float32),
            pltpu.VMEM((dgs, 8, _BL), jnp.float32),
            pltpu.SemaphoreType.DMA,
            pltpu.SemaphoreType.DMA,
            pltpu.SemaphoreType.DMA,
            pltpu.SemaphoreType.DMA,
        ],
        compiler_params=pltpu.CompilerParams(use_tc_tiling_on_sc=False,
                                             needs_layout_passes=False),
    )
    def sc_kernel(w_hbm, idx_hbm, pe_hbm, out_hbm,
                  idx_v, pe_v, g0, g1, o0, o1, sg0, sg1, sw0, sw1):
        wid = lax.axis_index("s") * 2 + lax.axis_index("c")
        pltpu.sync_copy(idx_hbm.at[wid], idx_v)
        pltpu.sync_copy(pe_hbm, pe_v)

        gb, ob = (g0, g1), (o0, o1)
        sg, sw = (sg0, sg1), (sw0, sw1)

        def slab_lg(s_):
            return s_ // n_bg, lax.rem(s_, n_bg)

        def issue_gather(k_, bi):
            pltpu.async_copy(w_hbm.at[idx_v.at[k_]], gb[bi], sg[bi])

        def wait_gather(k_, bi):
            pltpu.make_async_copy(w_hbm.at[idx_v.at[k_]], gb[bi],
                                  sg[bi]).wait()

        def issue_wb(k_, bi):
            l_, bg_ = slab_lg(wid * n_slabs + k_)
            for dg in range(dgs):
                pltpu.async_copy(ob[bi].at[dg], out_hbm.at[l_, dg, bg_],
                                 sw[bi])

        def wait_wb(k_, bi):
            l_, bg_ = slab_lg(wid * n_slabs + k_)
            for dg in range(dgs):
                pltpu.make_async_copy(ob[bi].at[dg], out_hbm.at[l_, dg, bg_],
                                      sw[bi]).wait()

        def compute(k_, bi):
            l_, _ = slab_lg(wid * n_slabs + k_)
            pes = [pe_v[l_, pl.ds(16 * j, _LANES)] for j in range(segs)]
            lane = lax.iota(jnp.int32, _LANES)
            dgv = [(16 * j + lane) >> 3 for j in range(segs)]
            dsv = [(16 * j + lane) & 7 for j in range(segs)]

            def row_body(bl, carry):
                blv = lax.broadcast_in_dim(bl, (_LANES,), ())
                for j in range(segs):
                    x = gb[bi][bl, pl.ds(16 * j, _LANES)]
                    y = x * scale + pes[j]
                    plsc.store_scatter(ob[bi], [dgv[j], dsv[j], blv], y)
                return carry

            lax.fori_loop(0, _BL, row_body, 0, unroll=4)

        for bi in range(_NBUF):
            issue_gather(bi, bi)
        for bi in range(_NBUF):
            wait_gather(bi, bi)
            compute(bi, bi)
            issue_gather(bi + _NBUF, bi)
            issue_wb(bi, bi)

        def group_body(g, carry):
            for bi in range(_NBUF):
                k_ = g * _NBUF + bi
                wait_gather(k_, bi)
                wait_wb(k_ - _NBUF, bi)
                compute(k_, bi)
                issue_gather(k_ + _NBUF, bi)
                issue_wb(k_, bi)
            return carry

        n_groups = n_slabs // _NBUF
        lax.fori_loop(1, n_groups - 1, group_body, 0)

        for bi in range(_NBUF):
            k_ = (n_groups - 1) * _NBUF + bi
            wait_gather(k_, bi)
            wait_wb(k_ - _NBUF, bi)
            compute(k_, bi)
            issue_wb(k_, bi)
        for bi in range(_NBUF):
            wait_wb((n_groups - 1) * _NBUF + bi, bi)

    return sc_kernel(W, idx, pe)


def kernel(inp, W):
    b, l = inp.shape
    d = W.shape[1]
    n_bg = b // _BL
    n_slabs = l * n_bg // _NUM_WORKERS
    idx = (inp.astype(jnp.int32).T.reshape(l * n_bg, _BL)
           .reshape(_NUM_WORKERS, n_slabs, _BL))
    pe = _positional_encoding(l, d)
    out5d = _run(idx, W, pe, b, l)
    return out5d.transpose(2, 4, 0, 1, 3).reshape(b, l, d)
